# baseline (speedup 1.0000x reference)
import functools

import jax
import jax.numpy as jnp
from jax import lax
from jax.experimental import pallas as pl
from jax.experimental.pallas import tpu as pltpu
from jax.experimental.pallas import tpu_sc as plsc

N_FIELDS = 26
FIELD_DIM = 100000
UNIFIED_DIM = 32
BATCH = 4096
TOTAL = BATCH * N_FIELDS
TOTAL_ROWS = N_FIELDS * FIELD_DIM
WIDE = 128

NUM_CORES = 2
NUM_SUBCORES = 16
LANES = 16
NW = NUM_CORES * NUM_SUBCORES
B_PER_W = TOTAL // NW
CHUNK = 128
N_CHUNKS = B_PER_W // CHUNK
VREGS_PER_W = B_PER_W // LANES
OFF_PERIOD = 208
OFF_VREGS = OFF_PERIOD // LANES

_mesh = plsc.VectorSubcoreMesh(core_axis_name="c", subcore_axis_name="s")


@functools.partial(
    pl.kernel,
    mesh=_mesh,
    out_type=jax.ShapeDtypeStruct((TOTAL, WIDE), jnp.float32),
    scratch_types=[
        pltpu.VMEM((B_PER_W,), jnp.int32),
        pltpu.VMEM((OFF_PERIOD,), jnp.int32),
        pltpu.VMEM((CHUNK, WIDE), jnp.float32),
        pltpu.SemaphoreType.DMA,
    ],
    compiler_params=pltpu.CompilerParams(use_tc_tiling_on_sc=True),
)
def _embed_gather(x_hbm, off_hbm, table_hbm, out_hbm, idx_v, off_v, wide_v, sem):
    wid = lax.axis_index("s") * NUM_CORES + lax.axis_index("c")
    base = wid * B_PER_W

    pltpu.sync_copy(x_hbm.at[pl.ds(base, B_PER_W)], idx_v)
    pltpu.sync_copy(off_hbm, off_v)

    def add_off(i, carry):
        sl = pl.ds(i * LANES, LANES)
        p = lax.rem(i, OFF_VREGS) * LANES
        idx_v[sl] = idx_v[sl] + off_v[pl.ds(p, LANES)]
        return carry

    lax.fori_loop(0, VREGS_PER_W, add_off, 0)

    def chunk_body(j, carry):
        c0 = j * CHUNK
        pltpu.async_copy(
            table_hbm.at[idx_v.at[pl.ds(c0, CHUNK)]], wide_v, sem
        ).wait()
        pltpu.sync_copy(wide_v, out_hbm.at[pl.ds(base + c0, CHUNK)])
        return carry

    lax.fori_loop(0, N_CHUNKS, chunk_body, 0)


def kernel(x_batch, W, embed_offsets):
    x_flat = x_batch.reshape(TOTAL)
    wide_table = jnp.pad(W, ((0, 0), (0, WIDE - UNIFIED_DIM)))
    off_row = jnp.concatenate(
        [jnp.zeros((1,), jnp.int32), embed_offsets.astype(jnp.int32)]
    )
    off_tile = jnp.tile(off_row, OFF_PERIOD // N_FIELDS)
    out = _embed_gather(x_flat, off_tile, wide_table)
    return out[:, :UNIFIED_DIM].reshape(BATCH, N_FIELDS * UNIFIED_DIM)


# speedup vs baseline: 1.0134x; 1.0134x over previous
import functools

import jax
import jax.numpy as jnp
from jax import lax
from jax.experimental import pallas as pl
from jax.experimental.pallas import tpu as pltpu
from jax.experimental.pallas import tpu_sc as plsc

N_FIELDS = 26
FIELD_DIM = 100000
UNIFIED_DIM = 32
BATCH = 4096
TOTAL = BATCH * N_FIELDS

NUM_CORES = 2
NUM_SUBCORES = 16
LANES = 16
NW = NUM_CORES * NUM_SUBCORES
B_PER_W = TOTAL // NW
CHUNK = 128
N_CHUNKS = B_PER_W // CHUNK
VREGS_PER_W = B_PER_W // LANES
OFF_PERIOD = 208
OFF_VREGS = OFF_PERIOD // LANES

_mesh = plsc.VectorSubcoreMesh(core_axis_name="c", subcore_axis_name="s")


@functools.partial(
    pl.kernel,
    mesh=_mesh,
    out_type=jax.ShapeDtypeStruct((TOTAL, UNIFIED_DIM), jnp.float32),
    scratch_types=[
        pltpu.VMEM((B_PER_W,), jnp.int32),
        pltpu.VMEM((OFF_PERIOD,), jnp.int32),
        pltpu.VMEM((B_PER_W, UNIFIED_DIM), jnp.float32),
        pltpu.SemaphoreType.DMA,
    ],
    compiler_params=pltpu.CompilerParams(use_tc_tiling_on_sc=False),
)
def _embed_gather(x_hbm, off_hbm, table_hbm, out_hbm, idx_v, off_v, rows_v, sem):
    wid = lax.axis_index("s") * NUM_CORES + lax.axis_index("c")
    base = wid * B_PER_W

    pltpu.sync_copy(x_hbm.at[pl.ds(base, B_PER_W)], idx_v)
    pltpu.sync_copy(off_hbm, off_v)

    def add_off(i, carry):
        sl = pl.ds(i * LANES, LANES)
        p = lax.rem(i, OFF_VREGS) * LANES
        idx_v[sl] = idx_v[sl] + off_v[pl.ds(p, LANES)]
        return carry

    lax.fori_loop(0, VREGS_PER_W, add_off, 0)

    def fire(j, carry):
        pltpu.async_copy(
            table_hbm.at[idx_v.at[pl.ds(j * CHUNK, CHUNK)]],
            rows_v.at[pl.ds(j * CHUNK, CHUNK)],
            sem,
        )
        return carry

    lax.fori_loop(0, N_CHUNKS, fire, 0)

    def drain(j, carry):
        pltpu.make_async_copy(
            table_hbm.at[idx_v.at[pl.ds(j * CHUNK, CHUNK)]],
            rows_v.at[pl.ds(j * CHUNK, CHUNK)],
            sem,
        ).wait()
        return carry

    lax.fori_loop(0, N_CHUNKS, drain, 0)

    pltpu.sync_copy(rows_v, out_hbm.at[pl.ds(base, B_PER_W)])


def kernel(x_batch, W, embed_offsets):
    x_flat = x_batch.reshape(TOTAL)
    off_row = jnp.concatenate(
        [jnp.zeros((1,), jnp.int32), embed_offsets.astype(jnp.int32)]
    )
    off_tile = jnp.tile(off_row, OFF_PERIOD // N_FIELDS)
    out = _embed_gather(x_flat, off_tile, W)
    return out.reshape(BATCH, N_FIELDS * UNIFIED_DIM)
